# baseline (speedup 1.0000x reference)
import functools

import jax
import jax.numpy as jnp
from jax import lax
from jax.experimental import pallas as pl
from jax.experimental.pallas import tpu as pltpu
from jax.experimental.pallas import tpu_sc as plsc

NB, NA, NC, NH, NW = 32, 5, 13, 19, 19
NCH = 19 + NC
NCELL = NH * NW
NT = 50
NBB = 1
E2M1 = 6.38905609893065
INV9E2M1 = 1.0 / (9.0 * E2M1)
INV9E2M1E = 1.0 / (9.0 * (E2M1 + 1e-5))
SIL_THRESH = 0.6
OBJECT_SCALE = 5.0


def _one_batch(pred_ref, tgt_ref, gx_ref, gy_ref, bi):
    t = tgt_ref[bi]

    g1 = t[:, 1]
    zero_ind = (g1 == 0.0).astype(jnp.float32)
    row = jax.lax.broadcasted_iota(jnp.int32, (NT, NT), 0)
    col = jax.lax.broadcasted_iota(jnp.int32, (NT, NT), 1)
    tri = (col <= row).astype(jnp.float32)
    zcount = jnp.sum(tri * zero_ind[None, :], axis=1)
    valid = zcount == 0.0
    valid_f = valid.astype(jnp.float32)

    noobj = jnp.zeros((), jnp.float32)
    m0 = None
    dense_parts = [(a * NCH, 0, NCELL) for a in range(NA - 1)]
    dense_parts.append(((NA - 1) * NCH, 176, NCELL - 176))
    for base, lo, w in dense_parts:
        conf_sum = jnp.zeros((NT, w), jnp.float32)
        for k in range(9):
            xraw = pred_ref[bi, base + 2 * k, lo:lo + w]
            yraw = pred_ref[bi, base + 2 * k + 1, lo:lo + w]
            if k == 0:
                xraw = jax.nn.sigmoid(xraw)
                yraw = jax.nn.sigmoid(yraw)
            px = (xraw + gx_ref[0, lo:lo + w]) * (16.0 / 19.0)
            py = (yraw + gy_ref[0, lo:lo + w]) * (12.0 / 19.0)
            gx = t[:, 1 + 2 * k] * 16.0
            gy = t[:, 2 + 2 * k] * 12.0
            dx = gx[:, None] - px[None, :]
            dy = gy[:, None] - py[None, :]
            s = dx * dx + dy * dy
            s1 = jnp.maximum(s, 1e-30)
            d40 = s1 * jax.lax.rsqrt(s1)
            e = jnp.exp2(2.8853900817779268 - d40 * 1.4426950408889634)
            conf_sum = conf_sum + jnp.maximum(e - 1.0, 0.0)
        confs = conf_sum * valid_f[:, None]
        cur = jnp.max(confs, axis=0) * INV9E2M1
        m = (cur <= SIL_THRESH).astype(jnp.float32)
        confsig = jax.nn.sigmoid(pred_ref[bi, base + 18, lo:lo + w])
        noobj = noobj + jnp.sum(m * confsig * confsig)
        if base == 0:
            m0 = m

    gi = jnp.clip((g1 * 19.0).astype(jnp.int32), 0, NW - 1)
    gj = jnp.clip((t[:, 2] * 19.0).astype(jnp.int32), 0, NH - 1)
    cell = gj * NW + gi
    same = (cell[:, None] == cell[None, :]) & valid[None, :] & (col > row)
    later_dup = jnp.sum(same.astype(jnp.float32), axis=1) > 0.0
    winner = (valid & jnp.logical_not(later_dup)).astype(jnp.float32)

    lane = jax.lax.broadcasted_iota(jnp.int32, (NT, NCELL), 1)
    onehot = (lane == cell[:, None]).astype(jnp.float32)
    vals0 = pred_ref[bi, 0:NCH, :]
    cls = vals0[19:NCH]
    mx = jnp.max(cls, axis=0)
    lse = mx + jnp.log(jnp.sum(jnp.exp(cls - mx[None, :]), axis=0))
    ext = jnp.concatenate([vals0, m0[None, :], lse[None, :]], axis=0)
    gathered = jax.lax.dot_general(
        onehot, ext, (((1,), (1,)), ((), ())),
        preferred_element_type=jnp.float32)

    gi_f = gi.astype(jnp.float32)
    gj_f = gj.astype(jnp.float32)
    coord = jnp.zeros((NT,), jnp.float32)
    cft_sum = jnp.zeros((NT,), jnp.float32)
    for k in range(9):
        xk = gathered[:, 2 * k]
        yk = gathered[:, 2 * k + 1]
        if k == 0:
            xk = jax.nn.sigmoid(xk)
            yk = jax.nn.sigmoid(yk)
        dxk = t[:, 1 + 2 * k] * 19.0 - gi_f - xk
        dyk = t[:, 2 + 2 * k] * 19.0 - gj_f - yk
        coord = coord + dxk * dxk + dyk * dyk
        sx = dxk * (16.0 / 19.0)
        sy = dyk * (12.0 / 19.0)
        s = sx * sx + sy * sy
        s1 = jnp.maximum(s, 1e-30)
        d40 = s1 * jax.lax.rsqrt(s1)
        cft_sum = cft_sum + jnp.maximum(jnp.exp(2.0 - d40) - 1.0, 0.0)
    cft = cft_sum * INV9E2M1E

    confg = jax.nn.sigmoid(gathered[:, 18])
    m0g = gathered[:, 32]
    lseg = gathered[:, 33]
    label = jnp.clip(t[:, 0].astype(jnp.int32), 0, NC - 1)
    lbl_oh = (jax.lax.broadcasted_iota(jnp.int32, (NT, NC), 1)
              == label[:, None]).astype(jnp.float32)
    logit_lbl = jnp.sum(lbl_oh * gathered[:, 19:NCH], axis=1)

    box = (0.5 * coord
           + 0.5 * OBJECT_SCALE * (confg - cft) ** 2
           - 0.5 * m0g * confg * confg
           + (lseg - logit_lbl))
    return 0.5 * noobj + jnp.sum(winner * box)


def _gather16(x, idx):
    dn = lax.GatherDimensionNumbers(
        offset_dims=(), collapsed_slice_dims=(0,), start_index_map=(0,))
    return lax.gather(x, idx[:, None], dn, slice_sizes=(1,),
                      mode=lax.GatherScatterMode.PROMISE_IN_BOUNDS)


def _sc_noobj_anchor4(pred2, tgt3, gxc, gyc):
    mesh = plsc.VectorSubcoreMesh(core_axis_name="c", subcore_axis_name="s")

    @functools.partial(
        pl.kernel,
        mesh=mesh,
        out_type=jax.ShapeDtypeStruct((NB * 16,), jnp.float32),
        scratch_types=[
            pltpu.VMEM((24, NCELL), jnp.float32),
            pltpu.VMEM((24, 64), jnp.float32),
            pltpu.VMEM((368,), jnp.float32),
            pltpu.VMEM((368,), jnp.float32),
            pltpu.VMEM((16,), jnp.float32),
        ],
    )
    def sc_kernel(pred_hbm, tgt_hbm, gxc_hbm, gyc_hbm, out_hbm,
                  pred_v, tgt_v, gxc_v, gyc_v, out_v):
        wid = lax.axis_index("s") * 2 + lax.axis_index("c")
        pltpu.sync_copy(pred_hbm.at[pl.ds(wid * (NA * NCH) + 4 * NCH, 24)],
                        pred_v)
        pltpu.sync_copy(tgt_hbm.at[pl.ds(wid * 24, 24)], tgt_v)
        pltpu.sync_copy(gxc_hbm, gxc_v)
        pltpu.sync_copy(gyc_hbm, gyc_v)

        for k in range(9):
            for v in range(4):
                tgt_v[1 + 2 * k, pl.ds(v * 16, 16)] = (
                    tgt_v[1 + 2 * k, pl.ds(v * 16, 16)] * 16.0)
                tgt_v[2 + 2 * k, pl.ds(v * 16, 16)] = (
                    tgt_v[2 + 2 * k, pl.ds(v * 16, 16)] * 12.0)

        lane = lax.broadcasted_iota(jnp.int32, (16,), 0)
        lanef = lane.astype(jnp.float32)

        fzf = jnp.full((16,), 99.0, jnp.float32)
        for v in range(4):
            g1 = tgt_v[1, pl.ds(v * 16, 16)]
            fzf = jnp.minimum(fzf, jnp.where(g1 == 0.0,
                                             lanef + (v * 16.0), 99.0))
        for sh in (1, 2, 4, 8):
            fzf = jnp.minimum(fzf, _gather16(fzf, (lane + sh) & 15))

        def chunk_body(c, acc):
            off = c * 16
            cf = jnp.full((16,), c, jnp.int32).astype(jnp.float32)
            okm = jnp.minimum(cf - 22.0, lanef - 9.0)
            okv = okm < 0.0
            pxs, pys = [], []
            for k in range(9):
                xr = jnp.where(okv, pred_v[2 * k, pl.ds(off, 16)], 0.0)
                yr = jnp.where(okv, pred_v[2 * k + 1, pl.ds(off, 16)], 0.0)
                if k == 0:
                    xr = 1.0 / (1.0 + jnp.exp(-xr))
                    yr = 1.0 / (1.0 + jnp.exp(-yr))
                pxs.append((xr + gxc_v[pl.ds(off, 16)]) * (16.0 / 19.0))
                pys.append((yr + gyc_v[pl.ds(off, 16)]) * (12.0 / 19.0))

            cur = jnp.zeros((16,), jnp.float32)
            for vb in range(4):
                gxb = [tgt_v[1 + 2 * k, pl.ds(vb * 16, 16)] for k in range(9)]
                gyb = [tgt_v[2 + 2 * k, pl.ds(vb * 16, 16)] for k in range(9)]

                def t_body(tl, cur, gxb=gxb, gyb=gyb, vb=vb):
                    tlv = jnp.full((16,), tl, jnp.int32)
                    tlf = tlv.astype(jnp.float32) + (vb * 16.0)
                    invs = jnp.where(tlf >= fzf, 1e9, 0.0)
                    conf = jnp.zeros((16,), jnp.float32)
                    for k in range(9):
                        dx = _gather16(gxb[k], tlv) + invs - pxs[k]
                        dy = _gather16(gyb[k], tlv) - pys[k]
                        s = dx * dx + dy * dy
                        i = lax.bitcast_convert_type(s, jnp.int32)
                        i = 0x5F3759DF - lax.shift_right_logical(i, 1)
                        r = lax.bitcast_convert_type(i, jnp.float32)
                        r = r * (1.5 - 0.5 * s * r * r)
                        r = r * (1.5 - 0.5 * s * r * r)
                        d40 = s * r
                        conf = conf + jnp.maximum(
                            jnp.exp(2.0 - d40) - 1.0, 0.0)
                    return jnp.maximum(cur, conf)

                cur = lax.fori_loop(0, 16, t_body, cur)
            c18 = jnp.where(okv, pred_v[18, pl.ds(off, 16)], 0.0)
            confsig = 1.0 / (1.0 + jnp.exp(-c18))
            mok = jnp.maximum(cur * INV9E2M1 - SIL_THRESH, okm + 0.5) <= 0.0
            return acc + jnp.where(mok, confsig * confsig, 0.0)

        acc = lax.fori_loop(0, 11, chunk_body,
                            jnp.zeros((16,), jnp.float32))
        out_v[pl.ds(0, 16)] = acc
        pltpu.sync_copy(out_v, out_hbm.at[pl.ds(wid * 16, 16)])

    return sc_kernel(pred2, tgt3, gxc, gyc)


def _region_loss_body(pred_ref, tgt_ref, gx_ref, gy_ref, out_ref):
    b = pl.program_id(0)
    acc = jnp.zeros((), jnp.float32)
    for bi in range(NBB):
        acc = acc + _one_batch(pred_ref, tgt_ref, gx_ref, gy_ref, bi)
    partial = acc * jnp.ones((1, 1), jnp.float32)

    @pl.when(b == 0)
    def _():
        out_ref[...] = partial

    @pl.when(b != 0)
    def _():
        out_ref[...] = out_ref[...] + partial


@functools.partial(jax.jit, static_argnames=())
def kernel(output, target):
    pred = output.reshape(NB, NA * NCH, NCELL)
    tgt = target.reshape(NB, NT, 21)
    gx = jnp.tile(jnp.arange(NW, dtype=jnp.float32)[None, :],
                  (NH, 1)).reshape(1, NCELL)
    gy = jnp.tile(jnp.arange(NH, dtype=jnp.float32)[:, None],
                  (1, NW)).reshape(1, NCELL)

    pred2 = output.reshape(NB * NA * NCH, NCELL)
    tgt3 = jnp.pad(tgt.transpose(0, 2, 1),
                   ((0, 0), (0, 3), (0, 14))).reshape(NB * 24, 64)
    gxc = jnp.pad(gx.reshape(-1), (0, 7))
    gyc = jnp.pad(gy.reshape(-1), (0, 7))
    sc_noobj = _sc_noobj_anchor4(pred2, tgt3, gxc, gyc)

    res = pl.pallas_call(
        _region_loss_body,
        grid=(NB // NBB,),
        in_specs=[
            pl.BlockSpec((NBB, NA * NCH, NCELL), lambda b: (b, 0, 0)),
            pl.BlockSpec((NBB, NT, 21), lambda b: (b, 0, 0)),
            pl.BlockSpec((1, NCELL), lambda b: (0, 0)),
            pl.BlockSpec((1, NCELL), lambda b: (0, 0)),
        ],
        out_specs=pl.BlockSpec((1, 1), lambda b: (0, 0)),
        out_shape=jax.ShapeDtypeStruct((1, 1), jnp.float32),
    )(pred, tgt, gx, gy)
    return res[0, 0] + 0.5 * jnp.sum(sc_noobj)


# speedup vs baseline: 1.4070x; 1.4070x over previous
import functools

import jax
import jax.numpy as jnp
from jax.experimental import pallas as pl

NB, NA, NC, NH, NW = 32, 5, 13, 19, 19
NCH = 19 + NC
NCELL = NH * NW
NT = 50
NBB = 1
E2M1 = 6.38905609893065
INV9E2M1 = 1.0 / (9.0 * E2M1)
INV9E2M1E = 1.0 / (9.0 * (E2M1 + 1e-5))
SIL_THRESH = 0.6
OBJECT_SCALE = 5.0


def _one_batch(pred_ref, tgt_ref, gx_ref, gy_ref, bi):
    t = tgt_ref[bi]

    g1 = t[:, 1]
    zero_ind = (g1 == 0.0).astype(jnp.float32)
    row = jax.lax.broadcasted_iota(jnp.int32, (NT, NT), 0)
    col = jax.lax.broadcasted_iota(jnp.int32, (NT, NT), 1)
    tri = (col <= row).astype(jnp.float32)
    zcount = jnp.sum(tri * zero_ind[None, :], axis=1)
    valid = zcount == 0.0
    valid_f = valid.astype(jnp.float32)

    noobj = jnp.zeros((), jnp.float32)
    m0 = None
    for a in range(NA):
        base = a * NCH
        conf_sum = jnp.zeros((NT, NCELL), jnp.float32)
        for k in range(9):
            xraw = pred_ref[bi, base + 2 * k, :]
            yraw = pred_ref[bi, base + 2 * k + 1, :]
            if k == 0:
                xraw = jax.nn.sigmoid(xraw)
                yraw = jax.nn.sigmoid(yraw)
            px = (xraw + gx_ref[0]) * (16.0 / 19.0)
            py = (yraw + gy_ref[0]) * (12.0 / 19.0)
            gx = t[:, 1 + 2 * k] * 16.0
            gy = t[:, 2 + 2 * k] * 12.0
            dx = gx[:, None] - px[None, :]
            dy = gy[:, None] - py[None, :]
            s = dx * dx + dy * dy
            s1 = jnp.maximum(s, 1e-30)
            d40 = s1 * jax.lax.rsqrt(s1)
            e = jnp.exp2(2.8853900817779268 - d40 * 1.4426950408889634)
            conf_sum = conf_sum + jnp.maximum(e - 1.0, 0.0)
        confs = conf_sum * valid_f[:, None]
        cur = jnp.max(confs, axis=0) * INV9E2M1
        m = (cur <= SIL_THRESH).astype(jnp.float32)
        confsig = jax.nn.sigmoid(pred_ref[bi, base + 18, :])
        noobj = noobj + jnp.sum(m * confsig * confsig)
        if a == 0:
            m0 = m

    gi = jnp.clip((g1 * 19.0).astype(jnp.int32), 0, NW - 1)
    gj = jnp.clip((t[:, 2] * 19.0).astype(jnp.int32), 0, NH - 1)
    cell = gj * NW + gi
    same = (cell[:, None] == cell[None, :]) & valid[None, :] & (col > row)
    later_dup = jnp.sum(same.astype(jnp.float32), axis=1) > 0.0
    winner = (valid & jnp.logical_not(later_dup)).astype(jnp.float32)

    lane = jax.lax.broadcasted_iota(jnp.int32, (NT, NCELL), 1)
    onehot = (lane == cell[:, None]).astype(jnp.float32)
    vals0 = pred_ref[bi, 0:NCH, :]
    cls = vals0[19:NCH]
    mx = jnp.max(cls, axis=0)
    lse = mx + jnp.log(jnp.sum(jnp.exp(cls - mx[None, :]), axis=0))
    ext = jnp.concatenate([vals0, m0[None, :], lse[None, :]], axis=0)
    gathered = jax.lax.dot_general(
        onehot, ext, (((1,), (1,)), ((), ())),
        preferred_element_type=jnp.float32)

    gi_f = gi.astype(jnp.float32)
    gj_f = gj.astype(jnp.float32)
    coord = jnp.zeros((NT,), jnp.float32)
    cft_sum = jnp.zeros((NT,), jnp.float32)
    for k in range(9):
        xk = gathered[:, 2 * k]
        yk = gathered[:, 2 * k + 1]
        if k == 0:
            xk = jax.nn.sigmoid(xk)
            yk = jax.nn.sigmoid(yk)
        dxk = t[:, 1 + 2 * k] * 19.0 - gi_f - xk
        dyk = t[:, 2 + 2 * k] * 19.0 - gj_f - yk
        coord = coord + dxk * dxk + dyk * dyk
        sx = dxk * (16.0 / 19.0)
        sy = dyk * (12.0 / 19.0)
        s = sx * sx + sy * sy
        s1 = jnp.maximum(s, 1e-30)
        d40 = s1 * jax.lax.rsqrt(s1)
        cft_sum = cft_sum + jnp.maximum(jnp.exp(2.0 - d40) - 1.0, 0.0)
    cft = cft_sum * INV9E2M1E

    confg = jax.nn.sigmoid(gathered[:, 18])
    m0g = gathered[:, 32]
    lseg = gathered[:, 33]
    label = jnp.clip(t[:, 0].astype(jnp.int32), 0, NC - 1)
    lbl_oh = (jax.lax.broadcasted_iota(jnp.int32, (NT, NC), 1)
              == label[:, None]).astype(jnp.float32)
    logit_lbl = jnp.sum(lbl_oh * gathered[:, 19:NCH], axis=1)

    box = (0.5 * coord
           + 0.5 * OBJECT_SCALE * (confg - cft) ** 2
           - 0.5 * m0g * confg * confg
           + (lseg - logit_lbl))
    return 0.5 * noobj + jnp.sum(winner * box)


def _region_loss_body(pred_ref, tgt_ref, gx_ref, gy_ref, out_ref):
    b = pl.program_id(0)
    acc = jnp.zeros((), jnp.float32)
    for bi in range(NBB):
        acc = acc + _one_batch(pred_ref, tgt_ref, gx_ref, gy_ref, bi)
    partial = acc * jnp.ones((1, 1), jnp.float32)

    @pl.when(b == 0)
    def _():
        out_ref[...] = partial

    @pl.when(b != 0)
    def _():
        out_ref[...] = out_ref[...] + partial


@functools.partial(jax.jit, static_argnames=())
def kernel(output, target):
    pred = output.reshape(NB, NA * NCH, NCELL)
    tgt = target.reshape(NB, NT, 21)
    gx = jnp.tile(jnp.arange(NW, dtype=jnp.float32)[None, :],
                  (NH, 1)).reshape(1, NCELL)
    gy = jnp.tile(jnp.arange(NH, dtype=jnp.float32)[:, None],
                  (1, NW)).reshape(1, NCELL)

    res = pl.pallas_call(
        _region_loss_body,
        grid=(NB // NBB,),
        in_specs=[
            pl.BlockSpec((NBB, NA * NCH, NCELL), lambda b: (b, 0, 0)),
            pl.BlockSpec((NBB, NT, 21), lambda b: (b, 0, 0)),
            pl.BlockSpec((1, NCELL), lambda b: (0, 0)),
            pl.BlockSpec((1, NCELL), lambda b: (0, 0)),
        ],
        out_specs=pl.BlockSpec((1, 1), lambda b: (0, 0)),
        out_shape=jax.ShapeDtypeStruct((1, 1), jnp.float32),
    )(pred, tgt, gx, gy)
    return res[0, 0]
